# Initial kernel scaffold; baseline (speedup 1.0000x reference)
import jax
import jax.numpy as jnp
from jax.experimental import pallas as pl


def kernel(qw_embed, dw_embed, inputs_qwm, inputs_dwm, w_uni, b_uni, w_bi, b_bi, w_tri, b_tri, w_dense, b_dense):
    raise NotImplementedError("write your pallas kernel here")


# speedup vs baseline: 4.6538x; 4.6538x over previous
import jax
import jax.numpy as jnp
from jax.experimental import pallas as pl
from jax.experimental.pallas import tpu as pltpu

_NK = 11


def _kernel_mus(n):
    mus = [1.0]
    bin_size = 2.0 / (n - 1)
    mus.append(1.0 - bin_size / 2.0)
    for i in range(1, n - 1):
        mus.append(mus[i] - bin_size)
    return mus


_MUS = _kernel_mus(_NK)
_SIGMAS = [0.001] + [0.1] * (_NK - 1)


def _body(xq_ref, xd_ref, w_ref, b_ref, mq_ref, dm_ref, wd_ref, bd_ref,
          feats_ref, score_ref):
    f32 = jnp.float32
    xq = xq_ref[0]
    xd = xd_ref[0]
    W = w_ref[...]
    bias = b_ref[...]
    Aq = jnp.dot(xq, W, preferred_element_type=f32) + bias
    Ad = jnp.dot(xd, W, preferred_element_type=f32) + bias

    def norml(x):
        x = jnp.maximum(x, 0.0) + 1e-9
        n = jnp.sqrt(jnp.sum(x * x, axis=1, keepdims=True))
        return x / jnp.maximum(n, 1e-10)

    nq = Aq.shape[0]
    qu = norml(Aq[:, 0:128])
    qb = norml(Aq[:, 128:256] + pltpu.roll(Aq[:, 256:384], nq - 1, axis=0))
    qt = norml(Aq[:, 384:512] + pltpu.roll(Aq[:, 512:640], nq - 1, axis=0)
               + pltpu.roll(Aq[:, 640:768], nq - 2, axis=0))
    Q = jnp.concatenate([qu, qb, qt], axis=0)

    nd = Ad.shape[0]
    du = norml(Ad[:, 0:128])
    db = norml(Ad[:, 128:256] + pltpu.roll(Ad[:, 256:384], nd - 1, axis=0))
    dt = norml(Ad[:, 384:512] + pltpu.roll(Ad[:, 512:640], nd - 1, axis=0)
               + pltpu.roll(Ad[:, 640:768], nd - 2, axis=0))

    dm = dm_ref[0]
    logs = []
    for h, dn in enumerate((du, db, dt)):
        S = jax.lax.dot_general(Q, dn, (((1,), (1,)), ((), ())),
                                preferred_element_type=f32)
        S = jnp.where(dm[h:h + 1, :] > 0.0, S, 1e6)
        cols = []
        for k in range(_NK):
            c = -0.5 / (_SIGMAS[k] * _SIGMAS[k])
            T = S - _MUS[k]
            P = jnp.exp(T * T * c)
            cols.append(jnp.sum(P, axis=1, keepdims=True))
        psum = jnp.concatenate(cols, axis=1)
        logs.append(jnp.log(jnp.maximum(psum, 1e-10)))
    L = jnp.concatenate(logs, axis=1)
    Mq = mq_ref[0]
    F = jnp.dot(Mq, L, preferred_element_type=f32)
    feats = jnp.concatenate([
        F[0:1, 0:11], F[0:1, 22:33], F[0:1, 11:22],
        F[1:2, 0:11], F[2:3, 0:11], F[1:2, 11:22],
        F[1:2, 22:33], F[2:3, 11:22], F[2:3, 22:33],
    ], axis=1)
    feats_ref[0] = feats
    score_ref[0] = (jnp.sum(feats * wd_ref[...], axis=1, keepdims=True)
                    + bd_ref[...])


@jax.jit
def kernel(qw_embed, dw_embed, inputs_qwm, inputs_dwm,
           w_uni, b_uni, w_bi, b_bi, w_tri, b_tri, w_dense, b_dense):
    f32 = jnp.float32
    B, LQ, D = qw_embed.shape
    LD = dw_embed.shape[1]
    C = w_uni.shape[0]

    Wcat = jnp.concatenate([
        w_uni[:, 0, 0, :],
        w_bi[:, 0, 0, :], w_bi[:, 0, 1, :],
        w_tri[:, 0, 0, :], w_tri[:, 0, 1, :], w_tri[:, 0, 2, :],
    ], axis=0).T
    z = jnp.zeros_like(b_bi)
    bcat = jnp.concatenate([b_uni, b_bi, z, b_tri, z, z])[None, :]

    iq = jnp.arange(LQ)
    Mq = jnp.zeros((B, 3, 3 * LQ), f32)
    Mq = Mq.at[:, 0, 0:LQ].set(inputs_qwm)
    Mq = Mq.at[:, 1, LQ:2 * LQ].set(inputs_qwm * (iq < LQ - 1))
    Mq = Mq.at[:, 2, 2 * LQ:3 * LQ].set(inputs_qwm * (iq < LQ - 2))

    idd = jnp.arange(LD)
    dmask = jnp.stack([
        inputs_dwm,
        inputs_dwm * (idd < LD - 1),
        inputs_dwm * (idd < LD - 2),
    ], axis=1)

    wd = w_dense.astype(f32)
    bd = b_dense.reshape(1, 1)

    feats3, score3 = pl.pallas_call(
        _body,
        grid=(B,),
        in_specs=[
            pl.BlockSpec((1, LQ, D), lambda b: (b, 0, 0)),
            pl.BlockSpec((1, LD, D), lambda b: (b, 0, 0)),
            pl.BlockSpec((D, 6 * C), lambda b: (0, 0)),
            pl.BlockSpec((1, 6 * C), lambda b: (0, 0)),
            pl.BlockSpec((1, 3, 3 * LQ), lambda b: (b, 0, 0)),
            pl.BlockSpec((1, 3, LD), lambda b: (b, 0, 0)),
            pl.BlockSpec((1, 9 * _NK), lambda b: (0, 0)),
            pl.BlockSpec((1, 1), lambda b: (0, 0)),
        ],
        out_specs=(
            pl.BlockSpec((1, 1, 9 * _NK), lambda b: (b, 0, 0)),
            pl.BlockSpec((1, 1, 1), lambda b: (b, 0, 0)),
        ),
        out_shape=(
            jax.ShapeDtypeStruct((B, 1, 9 * _NK), f32),
            jax.ShapeDtypeStruct((B, 1, 1), f32),
        ),
        compiler_params=pltpu.CompilerParams(
            dimension_semantics=("parallel",),
        ),
    )(qw_embed, dw_embed, Wcat, bcat, Mq, dmask, wd, bd)
    return score3.reshape(B), feats3.reshape(B, 9 * _NK)
